# Initial kernel scaffold; baseline (speedup 1.0000x reference)
import jax
import jax.numpy as jnp
from jax.experimental import pallas as pl


def kernel(x, edge_index, batch, W1, b1, W2, b2, W3, b3, Wfc, bfc):
    raise NotImplementedError("write your pallas kernel here")


# speedup vs baseline: 68.3365x; 68.3365x over previous
import functools
import jax
import jax.numpy as jnp
from jax import lax
from jax.experimental import pallas as pl
from jax.experimental.pallas import tpu as pltpu
from jax.experimental.pallas import tpu_sc as plsc

N = 100000
E = 6400000
G = 128

NC = 2
NS = 16
NW = NC * NS

NP = 100096
PAD = NP - N
RPW = NP // NW
RPS = NP // NS
GP = 136

EPC = E // NC
EPT = EPC // NS
EB = 2000
NSTEP = EPT // EB

_mesh = plsc.VectorSubcoreMesh(core_axis_name="c", subcore_axis_name="s")
_sc_params = pltpu.CompilerParams(use_tc_tiling_on_sc=False)


def _f32(*shape):
  return jax.ShapeDtypeStruct(shape, jnp.float32)


@functools.partial(
    pl.kernel,
    out_type=_f32(NC, NP, 8),
    mesh=_mesh,
    compiler_params=_sc_params,
    scratch_types=[
        pltpu.VMEM((EB,), jnp.int32),
        pltpu.VMEM((EB, 8), jnp.float32),
        pltpu.VMEM_SHARED((NP, 8), jnp.float32),
    ],
)
def _sc_deg(dst_hbm, ones_hbm, zeros_hbm, out_hbm, idx_v, ones_v, acc_sh):
  c = lax.axis_index("c")
  s = lax.axis_index("s")
  row0 = s * RPS
  pltpu.sync_copy(zeros_hbm.at[pl.ds(row0, RPS)], acc_sh.at[pl.ds(row0, RPS)])
  pltpu.sync_copy(ones_hbm, ones_v)
  plsc.subcore_barrier()
  ebase = c * EPC + s * EPT

  def step(i, carry):
    pltpu.sync_copy(dst_hbm.at[pl.ds(ebase + i * EB, EB)], idx_v)
    pltpu.sync_copy(ones_v, acc_sh.at[idx_v], add=True)
    return carry

  lax.fori_loop(0, NSTEP, step, 0)
  plsc.subcore_barrier()
  pltpu.sync_copy(acc_sh.at[pl.ds(row0, RPS)],
                  out_hbm.at[c, pl.ds(row0, RPS)])


def _make_prop(F, eb):
  nstep = EPT // eb
  assert nstep * eb == EPT and eb % 8 == 0 and F % 8 == 0

  @functools.partial(
      pl.kernel,
      out_type=_f32(NC, NP, F),
      mesh=_mesh,
      compiler_params=_sc_params,
      scratch_types=[
          pltpu.VMEM((eb,), jnp.int32),
          pltpu.VMEM((eb,), jnp.int32),
          pltpu.VMEM((eb, F), jnp.float32),
          pltpu.VMEM_SHARED((NP, F), jnp.float32),
      ],
  )
  def _prop(src_hbm, dst_hbm, g_hbm, zeros_hbm, out_hbm,
            idx_s, idx_d, rows_v, acc_sh):
    c = lax.axis_index("c")
    s = lax.axis_index("s")
    row0 = s * RPS
    pltpu.sync_copy(zeros_hbm.at[pl.ds(row0, RPS)],
                    acc_sh.at[pl.ds(row0, RPS)])
    plsc.subcore_barrier()
    ebase = c * EPC + s * EPT

    def step(i, carry):
      base = ebase + i * eb
      pltpu.sync_copy(src_hbm.at[pl.ds(base, eb)], idx_s)
      pltpu.sync_copy(dst_hbm.at[pl.ds(base, eb)], idx_d)
      pltpu.sync_copy(g_hbm.at[idx_s], rows_v)
      pltpu.sync_copy(rows_v, acc_sh.at[idx_d], add=True)
      return carry

    lax.fori_loop(0, nstep, step, 0)
    plsc.subcore_barrier()
    pltpu.sync_copy(acc_sh.at[pl.ds(row0, RPS)],
                    out_hbm.at[c, pl.ds(row0, RPS)])

  return _prop


_prop8 = _make_prop(8, 2000)
_prop16 = _make_prop(16, 1600)


@functools.partial(
    pl.kernel,
    out_type=_f32(NC, GP, 16),
    mesh=_mesh,
    compiler_params=_sc_params,
    scratch_types=[
        pltpu.VMEM((RPW,), jnp.int32),
        pltpu.VMEM((RPW, 16), jnp.float32),
        pltpu.VMEM_SHARED((GP, 16), jnp.float32),
    ],
)
def _sc_pool(h_hbm, batch_hbm, zeros_hbm, out_hbm, idx_v, rows_v, acc_sh):
  c = lax.axis_index("c")
  s = lax.axis_index("s")

  @pl.when(s == 0)
  def _():
    pltpu.sync_copy(zeros_hbm, acc_sh)

  plsc.subcore_barrier()
  row0 = (c * NS + s) * RPW
  pltpu.sync_copy(h_hbm.at[pl.ds(row0, RPW)], rows_v)
  pltpu.sync_copy(batch_hbm.at[pl.ds(row0, RPW)], idx_v)
  pltpu.sync_copy(rows_v, acc_sh.at[idx_v], add=True)
  plsc.subcore_barrier()

  @pl.when(s == 0)
  def _():
    pltpu.sync_copy(acc_sh, out_hbm.at[c])


_TCROWS = 3128
_TCGRID = NP // _TCROWS


def _rows_spec(f):
  return pl.BlockSpec((_TCROWS, f), lambda i: (i, 0))


def _full_spec(r, f):
  return pl.BlockSpec((r, f), lambda i: (0, 0))


def _tc1_body(d0, d1, x, dinv_o, gx_o):
  deg = d0[:, 0:1] + d1[:, 0:1] + 1.0
  dinv = lax.rsqrt(deg)
  dinv_o[...] = dinv
  gx_o[...] = x[...] * dinv


def _tc1(d0, d1, x):
  return pl.pallas_call(
      _tc1_body,
      grid=(_TCGRID,),
      in_specs=[_rows_spec(8), _rows_spec(8), _rows_spec(8)],
      out_specs=[_rows_spec(1), _rows_spec(8)],
      out_shape=[_f32(NP, 1), _f32(NP, 8)],
  )(d0, d1, x)


def _tc2_body(a0, a1, gx, dinv, W1, b1, W2, g2_o):
  p = dinv[...] * (a0[...] + a1[...] + gx[...])
  h1 = jnp.maximum(
      jnp.dot(p, W1[...], preferred_element_type=jnp.float32) + b1[...], 0.0)
  g2_o[...] = dinv[...] * jnp.dot(h1, W2[...],
                                  preferred_element_type=jnp.float32)


def _tc2(a0, a1, gx, dinv, W1p, b1, W2):
  return pl.pallas_call(
      _tc2_body,
      grid=(_TCGRID,),
      in_specs=[_rows_spec(8), _rows_spec(8), _rows_spec(8), _rows_spec(1),
                _full_spec(8, 32), _full_spec(1, 32), _full_spec(32, 16)],
      out_specs=_rows_spec(16),
      out_shape=_f32(NP, 16),
  )(a0, a1, gx, dinv, W1p, b1, W2)


def _tc3_body(a0, a1, g2, dinv, b2, W3, g3_o):
  h2 = jnp.maximum(dinv[...] * (a0[...] + a1[...] + g2[...]) + b2[...], 0.0)
  g3_o[...] = dinv[...] * jnp.dot(h2, W3[...],
                                  preferred_element_type=jnp.float32)


def _tc3(a0, a1, g2, dinv, b2, W3):
  return pl.pallas_call(
      _tc3_body,
      grid=(_TCGRID,),
      in_specs=[_rows_spec(16), _rows_spec(16), _rows_spec(16), _rows_spec(1),
                _full_spec(1, 16), _full_spec(16, 8)],
      out_specs=_rows_spec(8),
      out_shape=_f32(NP, 8),
  )(a0, a1, g2, dinv, b2, W3)


def _tc4_body(a0, a1, g3, dinv, b3, h_o):
  h3 = jnp.maximum(dinv[...] * (a0[...] + a1[...] + g3[...]) + b3[...], 0.0)
  ones = jnp.ones((_TCROWS, 1), jnp.float32)
  zeros = jnp.zeros((_TCROWS, 7), jnp.float32)
  h_o[...] = jnp.concatenate([h3, ones, zeros], axis=1)


def _tc4(a0, a1, g3, dinv, b3):
  return pl.pallas_call(
      _tc4_body,
      grid=(_TCGRID,),
      in_specs=[_rows_spec(8), _rows_spec(8), _rows_spec(8), _rows_spec(1),
                _full_spec(1, 8)],
      out_specs=_rows_spec(16),
      out_shape=_f32(NP, 16),
  )(a0, a1, g3, dinv, b3)


def _tc5_body(s0, s1, Wfc, bfc, out_o):
  acc = (s0[...] + s1[...])[:G]
  sums = acc[:, :8]
  cnts = jnp.maximum(acc[:, 8:9], 1.0)
  pooled = sums / cnts
  out_o[...] = jnp.dot(pooled, Wfc[...],
                       preferred_element_type=jnp.float32) + bfc[...]


def _tc5(s0, s1, Wfc, bfc):
  return pl.pallas_call(
      _tc5_body,
      out_shape=_f32(G, 3),
  )(s0, s1, Wfc, bfc)


@jax.jit
def kernel(x, edge_index, batch, W1, b1, W2, b2, W3, b3, Wfc, bfc):
  src = edge_index[0]
  dst = edge_index[1]
  x8 = jnp.pad(x, ((0, PAD), (0, 5)))
  batch_p = jnp.pad(batch, (0, PAD), constant_values=G)
  W1p = jnp.pad(W1, ((0, 5), (0, 0)))

  ones_eb = jnp.ones((EB, 8), jnp.float32)
  zeros16 = jnp.zeros((NP, 16), jnp.float32)
  zeros8 = jnp.zeros((NP, 8), jnp.float32)
  zgp = jnp.zeros((GP, 16), jnp.float32)

  deg_pp = _sc_deg(dst, ones_eb, zeros8)
  dinv, gx = _tc1(deg_pp[0], deg_pp[1], x8)

  aggx = _prop8(src, dst, gx, zeros8)
  g2 = _tc2(aggx[0], aggx[1], gx, dinv, W1p, b1.reshape(1, -1), W2)

  agg2 = _prop16(src, dst, g2, zeros16)
  g3 = _tc3(agg2[0], agg2[1], g2, dinv, b2.reshape(1, -1), W3)

  agg3 = _prop8(src, dst, g3, zeros8)
  h16 = _tc4(agg3[0], agg3[1], g3, dinv, b3.reshape(1, -1))

  sums_pp = _sc_pool(h16, batch_p, zgp)
  out = _tc5(sums_pp[0], sums_pp[1], Wfc, bfc.reshape(1, -1))
  return out
